# baseline (speedup 1.0000x reference)
import functools
import numpy as np
import jax
import jax.numpy as jnp
from jax import lax
from jax.experimental import pallas as pl
from jax.experimental.pallas import tpu as pltpu
from jax.experimental.pallas import tpu_sc as plsc

N = 10000
E = 320000
D = 128
H = 4
DH = 32
L = 4
ED = 5
VC = 8
G = 512

NC = 2
NS = 16
NW = NC * NS

PW1 = 128
PW2 = 32


def _sc_mesh():
    return plsc.VectorSubcoreMesh(
        core_axis_name="c", subcore_axis_name="s", num_cores=NC, num_subcores=NS
    )


def _sc_gather(table, idx2d, dw, cb=2):
    nb = idx2d.shape[0]
    ng = nb // cb
    iters = (ng + NW - 1) // NW
    it_outer = (iters + 1) // 2
    rpg = cb * 128

    @functools.partial(
        pl.kernel,
        out_type=jax.ShapeDtypeStruct((nb * 128, dw), jnp.float32),
        mesh=_sc_mesh(),
        compiler_params=pltpu.CompilerParams(use_tc_tiling_on_sc=False),
        scratch_types=[
            pltpu.VMEM((2, cb, 128), jnp.int32),
            pltpu.VMEM((2, rpg, dw), jnp.float32),
            pltpu.SemaphoreType.DMA,
            pltpu.SemaphoreType.DMA,
        ],
    )
    def k(table_h, idx_h, out_h, idxv, rows, gsem, wsem):
        wid = lax.axis_index("s") * NC + lax.axis_index("c")

        @pl.loop(0, it_outer)
        def _(ot):
            for b in range(2):
                it = ot * 2 + b
                g = lax.rem(it * NW + wid, ng)

                @pl.when(it >= 2)
                def _():
                    pltpu.make_async_copy(
                        table_h.at[pl.ds(0, rpg)], rows.at[b], wsem
                    ).wait()
                pltpu.sync_copy(idx_h.at[pl.ds(g * cb, cb)], idxv.at[b])
                descs = [
                    pltpu.async_copy(
                        table_h.at[idxv.at[b].at[j]],
                        rows.at[b].at[pl.ds(j * 128, 128)],
                        gsem,
                    )
                    for j in range(cb)
                ]
                for d_ in descs:
                    d_.wait()
                pltpu.async_copy(
                    rows.at[b], out_h.at[pl.ds(g * rpg, rpg)], wsem
                )
        for b in range(2):
            pltpu.make_async_copy(
                table_h.at[pl.ds(0, rpg)], rows.at[b], wsem
            ).wait()

    return k(table, idx2d)


def _sc_scatter(pay1, pay2, dst2d, zr1, zr2):
    nbt = dst2d.shape[0]
    half = nbt // NC
    iters = (half + NS - 1) // NS
    rpt = N // NS

    @functools.partial(
        pl.kernel,
        out_type=(
            jax.ShapeDtypeStruct((NC, N, PW1), jnp.float32),
            jax.ShapeDtypeStruct((NC, N, PW2), jnp.float32),
        ),
        mesh=_sc_mesh(),
        compiler_params=pltpu.CompilerParams(use_tc_tiling_on_sc=False),
        scratch_types=[
            pltpu.VMEM((128,), jnp.int32),
            pltpu.VMEM((128, PW1), jnp.float32),
            pltpu.VMEM((128, PW2), jnp.float32),
            pltpu.VMEM_SHARED((N, PW1), jnp.float32),
            pltpu.VMEM_SHARED((N, PW2), jnp.float32),
        ],
    )
    def k(p1_h, p2_h, dst_h, z1_h, z2_h, o1_h, o2_h,
          idxv, pb1, pb2, acc1, acc2):
        c = lax.axis_index("c")
        s = lax.axis_index("s")
        pltpu.sync_copy(z1_h, acc1.at[pl.ds(s * rpt, rpt)])
        pltpu.sync_copy(z2_h, acc2.at[pl.ds(s * rpt, rpt)])
        plsc.subcore_barrier()

        @pl.loop(0, iters)
        def _(it):
            r = it * NS + s

            @pl.when(r < half)
            def _():
                b = c * half + r
                pltpu.sync_copy(dst_h.at[b], idxv)
                pltpu.sync_copy(p1_h.at[pl.ds(b * 128, 128)], pb1)
                pltpu.sync_copy(p2_h.at[pl.ds(b * 128, 128)], pb2)
                pltpu.sync_copy(pb1, acc1.at[idxv], add=True)
                pltpu.sync_copy(pb2, acc2.at[idxv], add=True)

        plsc.subcore_barrier()
        pltpu.sync_copy(
            acc1.at[pl.ds(s * rpt, rpt)],
            o1_h.at[c].at[pl.ds(s * rpt, rpt)],
        )
        pltpu.sync_copy(
            acc2.at[pl.ds(s * rpt, rpt)],
            o2_h.at[c].at[pl.ds(s * rpt, rpt)],
        )

    return k(pay1, pay2, dst2d, zr1, zr2)


def _ln(x):
    m = jnp.mean(x, axis=-1, keepdims=True)
    d = x - m
    v = jnp.mean(d * d, axis=-1, keepdims=True)
    return d / jnp.sqrt(v + 1e-5)


def _dot(a, b):
    return jnp.dot(a, b, preferred_element_type=jnp.float32)


def _mlp2(x, w1, b1, w2, b2):
    h = jax.nn.relu(_ln(_dot(x, w1) + b1))
    return _dot(h, w2) + b2


def _full(shape):
    return pl.BlockSpec(shape, lambda i: tuple(0 for _ in shape))


def _embed_call(f8, xpad, w8, b, qw1, qb1, qw2, qb2, vecemb):
    bn = 1000
    grid = (N // bn,)

    def body(f_ref, x_ref, w_ref, b_ref, q1_ref, qb1_ref, q2_ref, qb2_ref,
             ve_ref, sq_ref, vec_ref):
        s0 = _dot(f_ref[...], w_ref[...]) + b_ref[...]
        q = _mlp2(s0, q1_ref[...], qb1_ref[...], q2_ref[...], qb2_ref[...])
        sq_ref[0] = s0
        sq_ref[1] = q
        x = x_ref[...]
        ve = ve_ref[...]
        for d in range(3):
            vec_ref[d] = x[:, d:d + 1] * ve

    return pl.pallas_call(
        body,
        grid=grid,
        in_specs=[
            pl.BlockSpec((bn, 8), lambda i: (i, 0)),
            pl.BlockSpec((bn, 16), lambda i: (i, 0)),
            _full((8, D)), _full((1, D)),
            _full((D, D)), _full((1, D)), _full((D, D)), _full((1, D)),
            _full((1, VC)),
        ],
        out_specs=[
            pl.BlockSpec((2, bn, D), lambda i: (0, i, 0)),
            pl.BlockSpec((3, bn, VC), lambda i: (0, i, 0)),
        ],
        out_shape=[
            jax.ShapeDtypeStruct((2, N, D), jnp.float32),
            jax.ShapeDtypeStruct((3, N, VC), jnp.float32),
        ],
    )(f8, xpad, w8, b, qw1, qb1, qw2, qb2, vecemb)


def _edge_call(gth, xg, ea8, kw):
    be = 1280
    nblk = E // be
    grid = (nblk,)
    scale = 1.0 / np.sqrt(DH)

    def body(ss_ref, qd_ref, xs_ref, xd_ref, ea_ref,
             kw1, kb1, kw2, kb2,
             vw1, vb1, vw2, vb2,
             gw, gb, msg_ref, rest_ref):
        rel = xs_ref[...] - xd_ref[...]
        d2 = jnp.sum(rel * rel, axis=-1, keepdims=True) + 1e-12
        dist = jnp.sqrt(d2)
        ru = rel / (dist + 1e-8)
        ss = ss_ref[...]
        ea = ea_ref[...]
        ein = jnp.concatenate(
            [ss, ea[:, :ED], dist, jnp.zeros((be, 2), jnp.float32)], axis=-1)
        k1 = _dot(ein, kw1[...]) + kb1[...]
        kk = _dot(jax.nn.relu(_ln(k1)), kw2[...]) + kb2[...]
        v1 = _dot(ein, vw1[...]) + vb1[...]
        vv = _dot(jax.nn.relu(_ln(v1)), vw2[...]) + vb2[...]
        qd = qd_ref[...]
        msgs = []
        exs = []
        for h in range(H):
            sl = slice(h * DH, (h + 1) * DH)
            lg = jnp.sum(qd[:, sl] * kk[:, sl], axis=-1, keepdims=True) * scale
            exh = jnp.exp(lg)
            exs.append(exh)
            msgs.append(exh * vv[:, sl])
        gate = jnp.tanh(_dot(vv, gw[...]) + gb[...])
        msg_ref[...] = jnp.concatenate(msgs, axis=-1)
        rest_ref[...] = jnp.concatenate(exs + [
            jnp.zeros((be, 4), jnp.float32),
            gate * ru[:, 0:1],
            gate * ru[:, 1:2],
            gate * ru[:, 2:3],
        ], axis=-1)

    return pl.pallas_call(
        body,
        grid=grid,
        in_specs=[
            pl.BlockSpec((be, D), lambda i: (i, 0)),
            pl.BlockSpec((be, D), lambda i, n=nblk: (i + n, 0)),
            pl.BlockSpec((be, 16), lambda i: (i, 0)),
            pl.BlockSpec((be, 16), lambda i, n=nblk: (i + n, 0)),
            pl.BlockSpec((be, 8), lambda i: (i, 0)),
            _full((D + 8, D)), _full((1, D)), _full((D, D)), _full((1, D)),
            _full((D + 8, D)), _full((1, D)), _full((D, D)), _full((1, D)),
            _full((D, VC)), _full((1, VC)),
        ],
        out_specs=[
            pl.BlockSpec((be, PW1), lambda i: (i, 0)),
            pl.BlockSpec((be, PW2), lambda i: (i, 0)),
        ],
        out_shape=[
            jax.ShapeDtypeStruct((E, PW1), jnp.float32),
            jax.ShapeDtypeStruct((E, PW2), jnp.float32),
        ],
    )(gth, gth, xg, xg, ea8, *kw)


def _node_update(parts1, parts2, sq, vec3, ow1, ob1, ow2, ob2, vscale,
                 hw1, hb1, hw2, hb2, last):
    bn = 1000
    grid = (N // bn,)

    def body(p1_ref, p2_ref, s_ref, v_ref, ow1r, ob1r, ow2r, ob2r, vsr,
             h1r, hb1r, h2r, hb2r, out_ref, vec_ref):
        num = p1_ref[0] + p1_ref[1]
        tbl2 = p2_ref[0] + p2_ref[1]
        aggs = []
        for h in range(H):
            denh = tbl2[:, h:h + 1]
            denh = jnp.where(denh > 0, denh, 1.0)
            aggs.append(num[:, h * DH:(h + 1) * DH] / denh)
        agg = jnp.concatenate(aggs, axis=-1)
        o = _mlp2(agg, ow1r[...], ob1r[...], ow2r[...], ob2r[...])
        s2 = _ln(s_ref[0] + o)
        vold = v_ref[...]
        vn = [vold[d] + tbl2[:, 8 + 8 * d:16 + 8 * d] for d in range(3)]
        n2 = vn[0] * vn[0] + vn[1] * vn[1] + vn[2] * vn[2] + 1e-12
        fac = vsr[...] / (1.0 + jnp.sqrt(n2))
        vf = [vn[d] * fac for d in range(3)]
        if last:
            vinv = jnp.sqrt(vf[0] * vf[0] + vf[1] * vf[1] + vf[2] * vf[2] + 1e-12)
            h1 = _dot(jnp.concatenate([s2, vinv], axis=-1), h1r[...]) + hb1r[...]
            feat = _dot(jax.nn.relu(_ln(h1)), h2r[...]) + hb2r[...] + s2
            out_ref[0] = feat
        else:
            q = _mlp2(s2, h1r[...], hb1r[...], h2r[...], hb2r[...])
            out_ref[0] = s2
            out_ref[1] = q
        for d in range(3):
            vec_ref[d] = vf[d]

    nout = 1 if last else 2
    return pl.pallas_call(
        body,
        grid=grid,
        in_specs=[
            pl.BlockSpec((2, bn, PW1), lambda i: (0, i, 0)),
            pl.BlockSpec((2, bn, PW2), lambda i: (0, i, 0)),
            pl.BlockSpec((1, bn, D), lambda i: (0, i, 0)),
            pl.BlockSpec((3, bn, VC), lambda i: (0, i, 0)),
            _full((D, D)), _full((1, D)), _full((D, D)), _full((1, D)),
            _full((1, VC)),
            _full((D + VC, D) if last else (D, D)), _full((1, D)),
            _full((D, D)), _full((1, D)),
        ],
        out_specs=[
            pl.BlockSpec((nout, bn, D), lambda i: (0, i, 0)),
            pl.BlockSpec((3, bn, VC), lambda i: (0, i, 0)),
        ],
        out_shape=[
            jax.ShapeDtypeStruct((nout, N, D), jnp.float32),
            jax.ShapeDtypeStruct((3, N, VC), jnp.float32),
        ],
    )(parts1, parts2, sq, vec3, ow1, ob1, ow2, ob2, vscale, hw1, hb1, hw2, hb2)


def _offsets_call(gids3, slt):
    bn = 1000
    grid = (N // bn,)

    def body(g_ref, slt_ref, oc_ref):
        i = pl.program_id(0)
        gcol = g_ref[0]
        m = (lax.broadcasted_iota(jnp.int32, (bn, G), 1) == gcol).astype(jnp.float32)
        cnt = jnp.sum(m, axis=0, keepdims=True)

        @pl.when(i == 0)
        def _():
            oc_ref[1:2, :] = cnt

        @pl.when(i > 0)
        def _():
            oc_ref[1:2, :] = oc_ref[1:2, :] + cnt

        @pl.when(i == grid[0] - 1)
        def _():
            counts = oc_ref[1:2, :]
            oc_ref[0:1, :] = _dot(counts, slt_ref[...])

    return pl.pallas_call(
        body,
        grid=grid,
        in_specs=[
            pl.BlockSpec((1, bn, 1), lambda i: (i, 0, 0)),
            _full((G, G)),
        ],
        out_specs=pl.BlockSpec((2, G), lambda i: (0, 0)),
        out_shape=jax.ShapeDtypeStruct((2, G), jnp.float32),
    )(gids3, slt)


def _pool_final_call(feat, oci, gmw1, gmb1, gmw2r, gmb2):
    def body(f_ref, oc_ref, w1, b1, w2r, b2, pooled_ref, fin_ref):
        g = pl.program_id(0)
        off = oc_ref[0, g]
        cnt = oc_ref[1, g]
        start = jnp.minimum(off, N - 128)
        rows = f_ref[pl.ds(start, 128), :]
        ridx = start + lax.broadcasted_iota(jnp.int32, (128, 1), 0)
        keep = (ridx >= off) & (ridx < off + cnt)
        mx = jnp.max(jnp.where(keep, rows, -3.4e38), axis=0, keepdims=True)
        pooled_ref[pl.ds(g, 1), :] = jnp.where(cnt > 0, mx, 0.0)

        @pl.when(g == G - 1)
        def _():
            p = pooled_ref[...]
            h = jax.nn.relu(_ln(_dot(p, w1[...]) + b1[...]))
            fin_ref[...] = _dot(h, w2r[...]) + b2[...]

    return pl.pallas_call(
        body,
        grid=(G,),
        in_specs=[
            _full((N, D)),
            pl.BlockSpec((2, G), lambda g: (0, 0), memory_space=pltpu.SMEM),
            _full((D, D)), _full((1, D)), _full((D, 1)), _full((1, 1)),
        ],
        out_specs=[
            pl.BlockSpec((G, D), lambda g: (0, 0)),
            pl.BlockSpec((G, 1), lambda g: (0, 0)),
        ],
        out_shape=[
            jax.ShapeDtypeStruct((G, D), jnp.float32),
            jax.ShapeDtypeStruct((G, 1), jnp.float32),
        ],
    )(feat, oci, gmw1, gmb1, gmw2r, gmb2)[1]


def _row(x):
    return x.reshape(1, -1)


def kernel(node_f, node_x, edge_index, edge_attr, graph_ids, params):
    p = params
    src = edge_index[0].astype(jnp.int32)
    dst = edge_index[1].astype(jnp.int32)

    f6 = node_f[..., 0]
    f8 = jnp.concatenate(
        [f6[:, :5], f6[:, 5:6] / 9.0, jnp.zeros((N, 2), jnp.float32)], axis=1)
    w8 = jnp.concatenate([p['embed_W'], jnp.zeros((2, D), jnp.float32)], axis=0)
    xpad = jnp.concatenate([node_x, jnp.zeros((N, 13), jnp.float32)], axis=1)
    ea8 = jnp.concatenate([edge_attr, jnp.zeros((E, 3), jnp.float32)], axis=1)
    idx2 = jnp.concatenate([src, dst + N]).reshape(2 * E // 128, 128)
    idxx = jnp.concatenate([src, dst]).reshape(2 * E // 128, 128)
    dst2d = dst.reshape(E // 128, 128)
    zr1 = jnp.zeros((N // NS, PW1), jnp.float32)
    zr2 = jnp.zeros((N // NS, PW2), jnp.float32)
    gids3 = graph_ids.astype(jnp.int32).reshape(N // 1000, 1000, 1)
    slt = jnp.triu(jnp.ones((G, G), jnp.float32), 1)

    xg = _sc_gather(xpad, idxx, 16)

    l0 = p['l0']
    sq, vec3 = _embed_call(
        f8, xpad, w8, _row(p['embed_b']),
        l0['qW1'], _row(l0['qb1']), l0['qW2'], _row(l0['qb2']),
        p['vec_embed'])

    feat = None
    for l in range(L):
        lp = p['l%d' % l]
        table = sq.reshape(2 * N, D)
        gth = _sc_gather(table, idx2, D)
        zpad2 = jnp.zeros((2, D), jnp.float32)
        kw = (
            jnp.concatenate([lp['kW1'], zpad2], 0), _row(lp['kb1']),
            lp['kW2'], _row(lp['kb2']),
            jnp.concatenate([lp['vW1'], zpad2], 0), _row(lp['vb1']),
            lp['vW2'], _row(lp['vb2']),
            lp['gW'], _row(lp['gb']),
        )
        pay1, pay2 = _edge_call(gth, xg, ea8, kw)
        parts1, parts2 = _sc_scatter(pay1, pay2, dst2d, zr1, zr2)
        if l < L - 1:
            nlp = p['l%d' % (l + 1)]
            sq, vec3 = _node_update(
                parts1, parts2, sq, vec3,
                lp['oW1'], _row(lp['ob1']), lp['oW2'], _row(lp['ob2']),
                _row(lp['vscale']),
                nlp['qW1'], _row(nlp['qb1']), nlp['qW2'], _row(nlp['qb2']),
                last=False)
        else:
            feat3, _ = _node_update(
                parts1, parts2, sq, vec3,
                lp['oW1'], _row(lp['ob1']), lp['oW2'], _row(lp['ob2']),
                _row(lp['vscale']),
                p['nmW1'], _row(p['nmb1']), p['nmW2'], _row(p['nmb2']),
                last=True)
            feat = feat3.reshape(N, D)

    oc = _offsets_call(gids3, slt)
    oci = oc.astype(jnp.int32)
    return _pool_final_call(
        feat, oci, p['gmW1'], _row(p['gmb1']), p['gmW2'],
        _row(p['gmb2']))


# speedup vs baseline: 1.0094x; 1.0094x over previous
import functools
import numpy as np
import jax
import jax.numpy as jnp
from jax import lax
from jax.experimental import pallas as pl
from jax.experimental.pallas import tpu as pltpu
from jax.experimental.pallas import tpu_sc as plsc

N = 10000
E = 320000
D = 128
H = 4
DH = 32
L = 4
ED = 5
VC = 8
G = 512

NC = 2
NS = 16
NW = NC * NS

PW1 = 128
PW2 = 32


def _sc_mesh():
    return plsc.VectorSubcoreMesh(
        core_axis_name="c", subcore_axis_name="s", num_cores=NC, num_subcores=NS
    )


def _sc_gather(table, idx2d, dw, cb=2):
    nb = idx2d.shape[0]
    ng = nb // cb
    iters = (ng + NW - 1) // NW
    it_outer = (iters + 1) // 2
    rpg = cb * 128

    @functools.partial(
        pl.kernel,
        out_type=jax.ShapeDtypeStruct((nb * 128, dw), jnp.float32),
        mesh=_sc_mesh(),
        compiler_params=pltpu.CompilerParams(use_tc_tiling_on_sc=False),
        scratch_types=[
            pltpu.VMEM((2, cb, 128), jnp.int32),
            pltpu.VMEM((2, rpg, dw), jnp.float32),
            pltpu.SemaphoreType.DMA,
            pltpu.SemaphoreType.DMA,
        ],
    )
    def k(table_h, idx_h, out_h, idxv, rows, gsem, wsem):
        wid = lax.axis_index("s") * NC + lax.axis_index("c")

        def grp(it):
            return lax.rem(it * NW + wid, ng)

        def stage_idx(it, b):
            pltpu.sync_copy(idx_h.at[pl.ds(grp(it) * cb, cb)], idxv.at[b])

        def fire(it, b):
            for j in range(cb):
                pltpu.async_copy(
                    table_h.at[idxv.at[b].at[j]],
                    rows.at[b].at[pl.ds(j * 128, 128)],
                    gsem,
                )

        def drain_g(b):
            pltpu.make_async_copy(
                table_h.at[pl.ds(0, rpg)], rows.at[b], gsem
            ).wait()

        def drain_w(b):
            pltpu.make_async_copy(
                table_h.at[pl.ds(0, rpg)], rows.at[b], wsem
            ).wait()

        stage_idx(0, 0)
        fire(0, 0)
        stage_idx(1, 1)

        @pl.loop(0, it_outer)
        def _(ot):
            for b in range(2):
                it = ot * 2 + b
                nb_ = 1 - b
                drain_g(b)
                pltpu.async_copy(
                    rows.at[b], out_h.at[pl.ds(grp(it) * rpg, rpg)], wsem
                )

                @pl.when(it >= 1)
                def _():
                    drain_w(nb_)
                fire(it + 1, nb_)
                stage_idx(it + 2, b)

        drain_g(0)
        drain_w(1)

    return k(table, idx2d)


def _sc_scatter(pay1, pay2, dst2d, zr1, zr2):
    nbt = dst2d.shape[0]
    half = nbt // NC
    iters = (half + NS - 1) // NS
    rpt = N // NS

    @functools.partial(
        pl.kernel,
        out_type=(
            jax.ShapeDtypeStruct((NC, N, PW1), jnp.float32),
            jax.ShapeDtypeStruct((NC, N, PW2), jnp.float32),
        ),
        mesh=_sc_mesh(),
        compiler_params=pltpu.CompilerParams(use_tc_tiling_on_sc=False),
        scratch_types=[
            pltpu.VMEM((128,), jnp.int32),
            pltpu.VMEM((128, PW1), jnp.float32),
            pltpu.VMEM((128, PW2), jnp.float32),
            pltpu.VMEM_SHARED((N, PW1), jnp.float32),
            pltpu.VMEM_SHARED((N, PW2), jnp.float32),
        ],
    )
    def k(p1_h, p2_h, dst_h, z1_h, z2_h, o1_h, o2_h,
          idxv, pb1, pb2, acc1, acc2):
        c = lax.axis_index("c")
        s = lax.axis_index("s")
        pltpu.sync_copy(z1_h, acc1.at[pl.ds(s * rpt, rpt)])
        pltpu.sync_copy(z2_h, acc2.at[pl.ds(s * rpt, rpt)])
        plsc.subcore_barrier()

        @pl.loop(0, iters)
        def _(it):
            r = it * NS + s

            @pl.when(r < half)
            def _():
                b = c * half + r
                pltpu.sync_copy(dst_h.at[b], idxv)
                pltpu.sync_copy(p1_h.at[pl.ds(b * 128, 128)], pb1)
                pltpu.sync_copy(p2_h.at[pl.ds(b * 128, 128)], pb2)
                pltpu.sync_copy(pb1, acc1.at[idxv], add=True)
                pltpu.sync_copy(pb2, acc2.at[idxv], add=True)

        plsc.subcore_barrier()
        pltpu.sync_copy(
            acc1.at[pl.ds(s * rpt, rpt)],
            o1_h.at[c].at[pl.ds(s * rpt, rpt)],
        )
        pltpu.sync_copy(
            acc2.at[pl.ds(s * rpt, rpt)],
            o2_h.at[c].at[pl.ds(s * rpt, rpt)],
        )

    return k(pay1, pay2, dst2d, zr1, zr2)


def _ln(x):
    m = jnp.mean(x, axis=-1, keepdims=True)
    d = x - m
    v = jnp.mean(d * d, axis=-1, keepdims=True)
    return d / jnp.sqrt(v + 1e-5)


def _dot(a, b):
    return jnp.dot(a, b, preferred_element_type=jnp.float32)


def _mlp2(x, w1, b1, w2, b2):
    h = jax.nn.relu(_ln(_dot(x, w1) + b1))
    return _dot(h, w2) + b2


def _full(shape):
    return pl.BlockSpec(shape, lambda i: tuple(0 for _ in shape))


def _embed_call(f8, xpad, w8, b, qw1, qb1, qw2, qb2, vecemb):
    bn = 1000
    grid = (N // bn,)

    def body(f_ref, x_ref, w_ref, b_ref, q1_ref, qb1_ref, q2_ref, qb2_ref,
             ve_ref, sq_ref, vec_ref):
        s0 = _dot(f_ref[...], w_ref[...]) + b_ref[...]
        q = _mlp2(s0, q1_ref[...], qb1_ref[...], q2_ref[...], qb2_ref[...])
        sq_ref[0] = s0
        sq_ref[1] = q
        x = x_ref[...]
        ve = ve_ref[...]
        for d in range(3):
            vec_ref[d] = x[:, d:d + 1] * ve

    return pl.pallas_call(
        body,
        grid=grid,
        in_specs=[
            pl.BlockSpec((bn, 8), lambda i: (i, 0)),
            pl.BlockSpec((bn, 16), lambda i: (i, 0)),
            _full((8, D)), _full((1, D)),
            _full((D, D)), _full((1, D)), _full((D, D)), _full((1, D)),
            _full((1, VC)),
        ],
        out_specs=[
            pl.BlockSpec((2, bn, D), lambda i: (0, i, 0)),
            pl.BlockSpec((3, bn, VC), lambda i: (0, i, 0)),
        ],
        out_shape=[
            jax.ShapeDtypeStruct((2, N, D), jnp.float32),
            jax.ShapeDtypeStruct((3, N, VC), jnp.float32),
        ],
    )(f8, xpad, w8, b, qw1, qb1, qw2, qb2, vecemb)


def _edge_call(gth, xg, ea8, kw):
    be = 1280
    nblk = E // be
    grid = (nblk,)
    scale = 1.0 / np.sqrt(DH)

    def body(ss_ref, qd_ref, xs_ref, xd_ref, ea_ref,
             kw1, kb1, kw2, kb2,
             vw1, vb1, vw2, vb2,
             gw, gb, msg_ref, rest_ref):
        rel = xs_ref[...] - xd_ref[...]
        d2 = jnp.sum(rel * rel, axis=-1, keepdims=True) + 1e-12
        dist = jnp.sqrt(d2)
        ru = rel / (dist + 1e-8)
        ss = ss_ref[...]
        ea = ea_ref[...]
        ein = jnp.concatenate(
            [ss, ea[:, :ED], dist, jnp.zeros((be, 2), jnp.float32)], axis=-1)
        k1 = _dot(ein, kw1[...]) + kb1[...]
        kk = _dot(jax.nn.relu(_ln(k1)), kw2[...]) + kb2[...]
        v1 = _dot(ein, vw1[...]) + vb1[...]
        vv = _dot(jax.nn.relu(_ln(v1)), vw2[...]) + vb2[...]
        qd = qd_ref[...]
        msgs = []
        exs = []
        for h in range(H):
            sl = slice(h * DH, (h + 1) * DH)
            lg = jnp.sum(qd[:, sl] * kk[:, sl], axis=-1, keepdims=True) * scale
            exh = jnp.exp(lg)
            exs.append(exh)
            msgs.append(exh * vv[:, sl])
        gate = jnp.tanh(_dot(vv, gw[...]) + gb[...])
        msg_ref[...] = jnp.concatenate(msgs, axis=-1)
        rest_ref[...] = jnp.concatenate(exs + [
            jnp.zeros((be, 4), jnp.float32),
            gate * ru[:, 0:1],
            gate * ru[:, 1:2],
            gate * ru[:, 2:3],
        ], axis=-1)

    return pl.pallas_call(
        body,
        grid=grid,
        in_specs=[
            pl.BlockSpec((be, D), lambda i: (i, 0)),
            pl.BlockSpec((be, D), lambda i, n=nblk: (i + n, 0)),
            pl.BlockSpec((be, 16), lambda i: (i, 0)),
            pl.BlockSpec((be, 16), lambda i, n=nblk: (i + n, 0)),
            pl.BlockSpec((be, 8), lambda i: (i, 0)),
            _full((D + 8, D)), _full((1, D)), _full((D, D)), _full((1, D)),
            _full((D + 8, D)), _full((1, D)), _full((D, D)), _full((1, D)),
            _full((D, VC)), _full((1, VC)),
        ],
        out_specs=[
            pl.BlockSpec((be, PW1), lambda i: (i, 0)),
            pl.BlockSpec((be, PW2), lambda i: (i, 0)),
        ],
        out_shape=[
            jax.ShapeDtypeStruct((E, PW1), jnp.float32),
            jax.ShapeDtypeStruct((E, PW2), jnp.float32),
        ],
    )(gth, gth, xg, xg, ea8, *kw)


def _node_update(parts1, parts2, sq, vec3, ow1, ob1, ow2, ob2, vscale,
                 hw1, hb1, hw2, hb2, last):
    bn = 1000
    grid = (N // bn,)

    def body(p1_ref, p2_ref, s_ref, v_ref, ow1r, ob1r, ow2r, ob2r, vsr,
             h1r, hb1r, h2r, hb2r, out_ref, vec_ref):
        num = p1_ref[0] + p1_ref[1]
        tbl2 = p2_ref[0] + p2_ref[1]
        aggs = []
        for h in range(H):
            denh = tbl2[:, h:h + 1]
            denh = jnp.where(denh > 0, denh, 1.0)
            aggs.append(num[:, h * DH:(h + 1) * DH] / denh)
        agg = jnp.concatenate(aggs, axis=-1)
        o = _mlp2(agg, ow1r[...], ob1r[...], ow2r[...], ob2r[...])
        s2 = _ln(s_ref[0] + o)
        vold = v_ref[...]
        vn = [vold[d] + tbl2[:, 8 + 8 * d:16 + 8 * d] for d in range(3)]
        n2 = vn[0] * vn[0] + vn[1] * vn[1] + vn[2] * vn[2] + 1e-12
        fac = vsr[...] / (1.0 + jnp.sqrt(n2))
        vf = [vn[d] * fac for d in range(3)]
        if last:
            vinv = jnp.sqrt(vf[0] * vf[0] + vf[1] * vf[1] + vf[2] * vf[2] + 1e-12)
            h1 = _dot(jnp.concatenate([s2, vinv], axis=-1), h1r[...]) + hb1r[...]
            feat = _dot(jax.nn.relu(_ln(h1)), h2r[...]) + hb2r[...] + s2
            out_ref[0] = feat
        else:
            q = _mlp2(s2, h1r[...], hb1r[...], h2r[...], hb2r[...])
            out_ref[0] = s2
            out_ref[1] = q
        for d in range(3):
            vec_ref[d] = vf[d]

    nout = 1 if last else 2
    return pl.pallas_call(
        body,
        grid=grid,
        in_specs=[
            pl.BlockSpec((2, bn, PW1), lambda i: (0, i, 0)),
            pl.BlockSpec((2, bn, PW2), lambda i: (0, i, 0)),
            pl.BlockSpec((1, bn, D), lambda i: (0, i, 0)),
            pl.BlockSpec((3, bn, VC), lambda i: (0, i, 0)),
            _full((D, D)), _full((1, D)), _full((D, D)), _full((1, D)),
            _full((1, VC)),
            _full((D + VC, D) if last else (D, D)), _full((1, D)),
            _full((D, D)), _full((1, D)),
        ],
        out_specs=[
            pl.BlockSpec((nout, bn, D), lambda i: (0, i, 0)),
            pl.BlockSpec((3, bn, VC), lambda i: (0, i, 0)),
        ],
        out_shape=[
            jax.ShapeDtypeStruct((nout, N, D), jnp.float32),
            jax.ShapeDtypeStruct((3, N, VC), jnp.float32),
        ],
    )(parts1, parts2, sq, vec3, ow1, ob1, ow2, ob2, vscale, hw1, hb1, hw2, hb2)


def _offsets_call(gids3, slt):
    bn = 1000
    grid = (N // bn,)

    def body(g_ref, slt_ref, oc_ref):
        i = pl.program_id(0)
        gcol = g_ref[0]
        m = (lax.broadcasted_iota(jnp.int32, (bn, G), 1) == gcol).astype(jnp.float32)
        cnt = jnp.sum(m, axis=0, keepdims=True)

        @pl.when(i == 0)
        def _():
            oc_ref[1:2, :] = cnt

        @pl.when(i > 0)
        def _():
            oc_ref[1:2, :] = oc_ref[1:2, :] + cnt

        @pl.when(i == grid[0] - 1)
        def _():
            counts = oc_ref[1:2, :]
            oc_ref[0:1, :] = _dot(counts, slt_ref[...])

    return pl.pallas_call(
        body,
        grid=grid,
        in_specs=[
            pl.BlockSpec((1, bn, 1), lambda i: (i, 0, 0)),
            _full((G, G)),
        ],
        out_specs=pl.BlockSpec((2, G), lambda i: (0, 0)),
        out_shape=jax.ShapeDtypeStruct((2, G), jnp.float32),
    )(gids3, slt)


def _pool_final_call(feat, oci, gmw1, gmb1, gmw2r, gmb2):
    def body(f_ref, oc_ref, w1, b1, w2r, b2, pooled_ref, fin_ref):
        g = pl.program_id(0)
        off = oc_ref[0, g]
        cnt = oc_ref[1, g]
        start = jnp.minimum(off, N - 128)
        rows = f_ref[pl.ds(start, 128), :]
        ridx = start + lax.broadcasted_iota(jnp.int32, (128, 1), 0)
        keep = (ridx >= off) & (ridx < off + cnt)
        mx = jnp.max(jnp.where(keep, rows, -3.4e38), axis=0, keepdims=True)
        pooled_ref[pl.ds(g, 1), :] = jnp.where(cnt > 0, mx, 0.0)

        @pl.when(g == G - 1)
        def _():
            p = pooled_ref[...]
            h = jax.nn.relu(_ln(_dot(p, w1[...]) + b1[...]))
            fin_ref[...] = _dot(h, w2r[...]) + b2[...]

    return pl.pallas_call(
        body,
        grid=(G,),
        in_specs=[
            _full((N, D)),
            pl.BlockSpec((2, G), lambda g: (0, 0), memory_space=pltpu.SMEM),
            _full((D, D)), _full((1, D)), _full((D, 1)), _full((1, 1)),
        ],
        out_specs=[
            pl.BlockSpec((G, D), lambda g: (0, 0)),
            pl.BlockSpec((G, 1), lambda g: (0, 0)),
        ],
        out_shape=[
            jax.ShapeDtypeStruct((G, D), jnp.float32),
            jax.ShapeDtypeStruct((G, 1), jnp.float32),
        ],
    )(feat, oci, gmw1, gmb1, gmw2r, gmb2)[1]


def _row(x):
    return x.reshape(1, -1)


def kernel(node_f, node_x, edge_index, edge_attr, graph_ids, params):
    p = params
    src = edge_index[0].astype(jnp.int32)
    dst = edge_index[1].astype(jnp.int32)

    f6 = node_f[..., 0]
    f8 = jnp.concatenate(
        [f6[:, :5], f6[:, 5:6] / 9.0, jnp.zeros((N, 2), jnp.float32)], axis=1)
    w8 = jnp.concatenate([p['embed_W'], jnp.zeros((2, D), jnp.float32)], axis=0)
    xpad = jnp.concatenate([node_x, jnp.zeros((N, 13), jnp.float32)], axis=1)
    ea8 = jnp.concatenate([edge_attr, jnp.zeros((E, 3), jnp.float32)], axis=1)
    idx2 = jnp.concatenate([src, dst + N]).reshape(2 * E // 128, 128)
    idxx = jnp.concatenate([src, dst]).reshape(2 * E // 128, 128)
    dst2d = dst.reshape(E // 128, 128)
    zr1 = jnp.zeros((N // NS, PW1), jnp.float32)
    zr2 = jnp.zeros((N // NS, PW2), jnp.float32)
    gids3 = graph_ids.astype(jnp.int32).reshape(N // 1000, 1000, 1)
    slt = jnp.triu(jnp.ones((G, G), jnp.float32), 1)

    xg = _sc_gather(xpad, idxx, 16)

    l0 = p['l0']
    sq, vec3 = _embed_call(
        f8, xpad, w8, _row(p['embed_b']),
        l0['qW1'], _row(l0['qb1']), l0['qW2'], _row(l0['qb2']),
        p['vec_embed'])

    feat = None
    for l in range(L):
        lp = p['l%d' % l]
        table = sq.reshape(2 * N, D)
        gth = _sc_gather(table, idx2, D)
        zpad2 = jnp.zeros((2, D), jnp.float32)
        kw = (
            jnp.concatenate([lp['kW1'], zpad2], 0), _row(lp['kb1']),
            lp['kW2'], _row(lp['kb2']),
            jnp.concatenate([lp['vW1'], zpad2], 0), _row(lp['vb1']),
            lp['vW2'], _row(lp['vb2']),
            lp['gW'], _row(lp['gb']),
        )
        pay1, pay2 = _edge_call(gth, xg, ea8, kw)
        parts1, parts2 = _sc_scatter(pay1, pay2, dst2d, zr1, zr2)
        if l < L - 1:
            nlp = p['l%d' % (l + 1)]
            sq, vec3 = _node_update(
                parts1, parts2, sq, vec3,
                lp['oW1'], _row(lp['ob1']), lp['oW2'], _row(lp['ob2']),
                _row(lp['vscale']),
                nlp['qW1'], _row(nlp['qb1']), nlp['qW2'], _row(nlp['qb2']),
                last=False)
        else:
            feat3, _ = _node_update(
                parts1, parts2, sq, vec3,
                lp['oW1'], _row(lp['ob1']), lp['oW2'], _row(lp['ob2']),
                _row(lp['vscale']),
                p['nmW1'], _row(p['nmb1']), p['nmW2'], _row(p['nmb2']),
                last=True)
            feat = feat3.reshape(N, D)

    oc = _offsets_call(gids3, slt)
    oci = oc.astype(jnp.int32)
    return _pool_final_call(
        feat, oci, p['gmW1'], _row(p['gmb1']), p['gmW2'],
        _row(p['gmb2']))
